# baseline (speedup 1.0000x reference)
import functools

import jax
import jax.numpy as jnp
from jax.experimental import pallas as pl

NUM_TOKENS = 16384
HIDDEN = 2048
EXPERTS = 64
BLOCK = 512


def _router_block(x_ref, wt_ref, w_out_ref, idx_out_ref):
    x = x_ref[...]
    wt = wt_ref[...]
    logits = jax.lax.dot_general(
        x, wt, (((1,), (0,)), ((), ())),
        preferred_element_type=jnp.float32)
    m = jnp.max(logits, axis=1, keepdims=True)
    s = jnp.sum(jnp.exp(logits - m), axis=1, keepdims=True)
    lane = jax.lax.broadcasted_iota(jnp.int32, logits.shape, 1)
    idx = jnp.min(jnp.where(logits == m, lane, EXPERTS), axis=1, keepdims=True)
    w_out_ref[...] = 1.0 / s
    idx_out_ref[...] = idx


@functools.partial(jax.jit, static_argnames=())
def kernel(hidden_states, W_gate):
    wt = W_gate.T
    n_blocks = NUM_TOKENS // BLOCK
    weights, indices = pl.pallas_call(
        _router_block,
        grid=(n_blocks,),
        in_specs=[
            pl.BlockSpec((BLOCK, HIDDEN), lambda i: (i, 0)),
            pl.BlockSpec((HIDDEN, EXPERTS), lambda i: (0, 0)),
        ],
        out_specs=[
            pl.BlockSpec((BLOCK, 1), lambda i: (i, 0)),
            pl.BlockSpec((BLOCK, 1), lambda i: (i, 0)),
        ],
        out_shape=[
            jax.ShapeDtypeStruct((NUM_TOKENS, 1), jnp.float32),
            jax.ShapeDtypeStruct((NUM_TOKENS, 1), jnp.int32),
        ],
    )(hidden_states, wt)
    return weights, indices.astype(jnp.int64)


# speedup vs baseline: 1.1416x; 1.1416x over previous
import functools

import jax
import jax.numpy as jnp
from jax.experimental import pallas as pl

NUM_TOKENS = 16384
HIDDEN = 2048
EXPERTS = 64
BLOCK = 1024


def _router_block(x_ref, wt_ref, w_out_ref, idx_out_ref):
    x = x_ref[...]
    wt = wt_ref[...]
    logits = jax.lax.dot_general(
        x, wt, (((1,), (0,)), ((), ())),
        preferred_element_type=jnp.float32)
    m = jnp.max(logits, axis=1, keepdims=True)
    s = jnp.sum(jnp.exp(logits - m), axis=1, keepdims=True)
    lane = jax.lax.broadcasted_iota(jnp.int32, logits.shape, 1)
    idx = jnp.min(jnp.where(logits == m, lane, EXPERTS), axis=1, keepdims=True)
    w_out_ref[...] = 1.0 / s
    idx_out_ref[...] = idx


@functools.partial(jax.jit, static_argnames=())
def kernel(hidden_states, W_gate):
    wt = W_gate.T
    n_blocks = NUM_TOKENS // BLOCK
    weights, indices = pl.pallas_call(
        _router_block,
        grid=(n_blocks,),
        in_specs=[
            pl.BlockSpec((BLOCK, HIDDEN), lambda i: (i, 0)),
            pl.BlockSpec((HIDDEN, EXPERTS), lambda i: (0, 0)),
        ],
        out_specs=[
            pl.BlockSpec((BLOCK, 1), lambda i: (i, 0)),
            pl.BlockSpec((BLOCK, 1), lambda i: (i, 0)),
        ],
        out_shape=[
            jax.ShapeDtypeStruct((NUM_TOKENS, 1), jnp.float32),
            jax.ShapeDtypeStruct((NUM_TOKENS, 1), jnp.int32),
        ],
    )(hidden_states, wt)
    return weights, indices.astype(jnp.int64)


# speedup vs baseline: 1.1718x; 1.0265x over previous
import functools

import jax
import jax.numpy as jnp
from jax.experimental import pallas as pl

NUM_TOKENS = 16384
HIDDEN = 2048
EXPERTS = 64
BLOCK = 2048


def _router_block(x_ref, wt_ref, w_out_ref, idx_out_ref):
    x = x_ref[...]
    wt = wt_ref[...]
    logits = jax.lax.dot_general(
        x, wt, (((1,), (0,)), ((), ())),
        preferred_element_type=jnp.float32)
    m = jnp.max(logits, axis=1, keepdims=True)
    s = jnp.sum(jnp.exp(logits - m), axis=1, keepdims=True)
    lane = jax.lax.broadcasted_iota(jnp.int32, logits.shape, 1)
    idx = jnp.min(jnp.where(logits == m, lane, EXPERTS), axis=1, keepdims=True)
    w_out_ref[...] = 1.0 / s
    idx_out_ref[...] = idx


@functools.partial(jax.jit, static_argnames=())
def kernel(hidden_states, W_gate):
    wt = W_gate.T
    n_blocks = NUM_TOKENS // BLOCK
    weights, indices = pl.pallas_call(
        _router_block,
        grid=(n_blocks,),
        in_specs=[
            pl.BlockSpec((BLOCK, HIDDEN), lambda i: (i, 0)),
            pl.BlockSpec((HIDDEN, EXPERTS), lambda i: (0, 0)),
        ],
        out_specs=[
            pl.BlockSpec((BLOCK, 1), lambda i: (i, 0)),
            pl.BlockSpec((BLOCK, 1), lambda i: (i, 0)),
        ],
        out_shape=[
            jax.ShapeDtypeStruct((NUM_TOKENS, 1), jnp.float32),
            jax.ShapeDtypeStruct((NUM_TOKENS, 1), jnp.int32),
        ],
    )(hidden_states, wt)
    return weights, indices.astype(jnp.int64)
